# Initial kernel scaffold; baseline (speedup 1.0000x reference)
import jax
import jax.numpy as jnp
from jax.experimental import pallas as pl


def kernel(x, edge_index, edge_attr, W1, b1, W2, b2, W3, b3):
    raise NotImplementedError("write your pallas kernel here")


# speedup vs baseline: 123.3549x; 123.3549x over previous
import functools

import jax
import jax.numpy as jnp
from jax import lax
from jax.experimental import pallas as pl
from jax.experimental.pallas import tpu as pltpu
from jax.experimental.pallas import tpu_sc as plsc

N = 50000
E = 1600000
NPAD = 50176
NC, NS = 2, 16
NTILES = NC * NS
EPT = E // NTILES
W = 2000
NWIN = EPT // W
SLICE = NPAD // NS


def _sc_mesh():
    return plsc.VectorSubcoreMesh(
        core_axis_name="c", subcore_axis_name="s", num_cores=NC, num_subcores=NS
    )


def _zero_slice(zbuf, sp_refs, sid):
    def zb(i, _):
        zbuf[pl.ds(i * 16, 16)] = jnp.zeros((16,), jnp.float32)
        return _
    lax.fori_loop(0, SLICE // 16, zb, None)
    for sp in sp_refs:
        pltpu.sync_copy(zbuf, sp.at[pl.ds(sid * SLICE, SLICE)])


def _deg_body(dst_hbm, w_hbm, out_hbm, dst_v, w_v, zbuf, deg_sp):
    cid = lax.axis_index("c")
    sid = lax.axis_index("s")
    wid = cid * NS + sid
    _zero_slice(zbuf, [deg_sp], sid)
    plsc.subcore_barrier()

    def win(j, _):
        base = wid * EPT + j * W
        pltpu.sync_copy(dst_hbm.at[pl.ds(base, W)], dst_v)
        pltpu.sync_copy(w_hbm.at[pl.ds(base, W)], w_v)
        pltpu.sync_copy(w_v, deg_sp.at[dst_v], add=True)
        return _

    lax.fori_loop(0, NWIN, win, None)
    plsc.subcore_barrier()
    sl = pl.ds(sid * SLICE, SLICE)
    pltpu.sync_copy(deg_sp.at[sl], zbuf)
    pltpu.sync_copy(zbuf, out_hbm.at[pl.ds(cid * NPAD + sid * SLICE, SLICE)])


_deg_call = functools.partial(
    pl.kernel,
    out_type=jax.ShapeDtypeStruct((NC * NPAD,), jnp.float32),
    mesh=_sc_mesh(),
    scratch_types=[
        pltpu.VMEM((W,), jnp.int32),
        pltpu.VMEM((W,), jnp.float32),
        pltpu.VMEM((SLICE,), jnp.float32),
        pltpu.VMEM_SHARED((NPAD,), jnp.float32),
    ],
)(_deg_body)


def _k2_body(degp_ref, xt_ref, dinv_ref, yt_ref):
    deg = degp_ref[0:1, :] + degp_ref[1:2, :] + 1.0
    dinv = jnp.where(deg > 0.0, lax.rsqrt(deg), 0.0)
    dinv_ref[...] = dinv
    yt_ref[...] = xt_ref[...] * dinv


_k2_call = pl.pallas_call(
    _k2_body,
    out_shape=[
        jax.ShapeDtypeStruct((1, NPAD), jnp.float32),
        jax.ShapeDtypeStruct((4, NPAD), jnp.float32),
    ],
)


def _agg_body(src_hbm, dst_hbm, w_hbm, yt_hbm, out_hbm,
              src_v, dst_v, w_v, r0, r1, r2, r3, zbuf,
              y0, y1, y2, y3, a0, a1, a2, a3):
    cid = lax.axis_index("c")
    sid = lax.axis_index("s")
    wid = cid * NS + sid
    r = [r0, r1, r2, r3]
    y = [y0, y1, y2, y3]
    a = [a0, a1, a2, a3]
    sl = pl.ds(sid * SLICE, SLICE)
    for k in range(4):
        pltpu.sync_copy(yt_hbm.at[pl.ds(k * NPAD + sid * SLICE, SLICE)], zbuf)
        pltpu.sync_copy(zbuf, y[k].at[sl])
    _zero_slice(zbuf, a, sid)
    plsc.subcore_barrier()

    def win(j, _):
        base = wid * EPT + j * W
        pltpu.sync_copy(src_hbm.at[pl.ds(base, W)], src_v)
        pltpu.sync_copy(dst_hbm.at[pl.ds(base, W)], dst_v)
        pltpu.sync_copy(w_hbm.at[pl.ds(base, W)], w_v)
        for k in range(4):
            pltpu.sync_copy(y[k].at[src_v], r[k])

        def mul(i, _c):
            off = pl.ds(i * 16, 16)
            wv = w_v[off]
            for k in range(4):
                r[k][off] = r[k][off] * wv
            return _c

        lax.fori_loop(0, W // 16, mul, None)
        for k in range(4):
            pltpu.sync_copy(r[k], a[k].at[dst_v], add=True)
        return _

    lax.fori_loop(0, NWIN, win, None)
    plsc.subcore_barrier()
    for k in range(4):
        pltpu.sync_copy(a[k].at[sl], zbuf)
        pltpu.sync_copy(zbuf, out_hbm.at[pl.ds((cid * 4 + k) * NPAD + sid * SLICE, SLICE)])


_agg_call = functools.partial(
    pl.kernel,
    out_type=jax.ShapeDtypeStruct((NC * 4 * NPAD,), jnp.float32),
    mesh=_sc_mesh(),
    scratch_types=[
        pltpu.VMEM((W,), jnp.int32),
        pltpu.VMEM((W,), jnp.int32),
        pltpu.VMEM((W,), jnp.float32),
        pltpu.VMEM((W,), jnp.float32),
        pltpu.VMEM((W,), jnp.float32),
        pltpu.VMEM((W,), jnp.float32),
        pltpu.VMEM((W,), jnp.float32),
        pltpu.VMEM((SLICE,), jnp.float32),
        pltpu.VMEM_SHARED((NPAD,), jnp.float32),
        pltpu.VMEM_SHARED((NPAD,), jnp.float32),
        pltpu.VMEM_SHARED((NPAD,), jnp.float32),
        pltpu.VMEM_SHARED((NPAD,), jnp.float32),
        pltpu.VMEM_SHARED((NPAD,), jnp.float32),
        pltpu.VMEM_SHARED((NPAD,), jnp.float32),
        pltpu.VMEM_SHARED((NPAD,), jnp.float32),
        pltpu.VMEM_SHARED((NPAD,), jnp.float32),
    ],
)(_agg_body)


BN = 2048


def _k4_body(aggp_ref, dinv_ref, yt_ref, wc_ref, bc_ref, out_ref):
    pre = (aggp_ref[0] + aggp_ref[1] + yt_ref[...]) * dinv_ref[...]
    out_ref[...] = (
        lax.dot_general(pre, wc_ref[...], (((0,), (0,)), ((), ())),
                        preferred_element_type=jnp.float32)
        + bc_ref[...]
    )


_k4_call = pl.pallas_call(
    _k4_body,
    grid=(pl.cdiv(N, BN),),
    in_specs=[
        pl.BlockSpec((NC, 4, BN), lambda i: (0, 0, i)),
        pl.BlockSpec((1, BN), lambda i: (0, i)),
        pl.BlockSpec((4, BN), lambda i: (0, i)),
        pl.BlockSpec((4, 48), lambda i: (0, 0)),
        pl.BlockSpec((1, 48), lambda i: (0, 0)),
    ],
    out_specs=pl.BlockSpec((BN, 48), lambda i: (i, 0)),
    out_shape=jax.ShapeDtypeStruct((N, 48), jnp.float32),
)


def kernel(x, edge_index, edge_attr, W1, b1, W2, b2, W3, b3):
    src = edge_index[0]
    dst = edge_index[1]
    w = edge_attr[:, 0]
    xt = jnp.pad(x.T, ((0, 0), (0, NPAD - N)))
    wc = jnp.concatenate([W1, W2, W3], axis=1)
    bc = jnp.concatenate([b1, b2, b3])[None, :]

    degp = _deg_call(dst, w).reshape(NC, NPAD)
    dinv, yt = _k2_call(degp, xt)
    aggp = _agg_call(src, dst, w, yt.reshape(4 * NPAD)).reshape(NC, 4, NPAD)
    return _k4_call(aggp, dinv, yt, wc, bc)


# speedup vs baseline: 159.7065x; 1.2947x over previous
import functools

import jax
import jax.numpy as jnp
from jax import lax
from jax.experimental import pallas as pl
from jax.experimental.pallas import tpu as pltpu
from jax.experimental.pallas import tpu_sc as plsc

N = 50000
E = 1600000
NPAD = 50176
NC, NS = 2, 16
NTILES = NC * NS
EPT = E // NTILES
W = 2000
NWIN = EPT // W
SLICE = NPAD // NS


def _sc_mesh():
    return plsc.VectorSubcoreMesh(
        core_axis_name="c", subcore_axis_name="s", num_cores=NC, num_subcores=NS
    )


def _zero_slice(zbuf, sp_refs, sid):
    def zb(i, _):
        zbuf[pl.ds(i * 16, 16)] = jnp.zeros((16,), jnp.float32)
        return _
    lax.fori_loop(0, SLICE // 16, zb, None)
    for sp in sp_refs:
        pltpu.sync_copy(zbuf, sp.at[pl.ds(sid * SLICE, SLICE)])


NB = 4


def _deg_body(dst_hbm, w_hbm, out_hbm, dst_v, w_v, zbuf, deg_sp,
              sem_l, sem_s):
    cid = lax.axis_index("c")
    sid = lax.axis_index("s")
    wid = cid * NS + sid
    _zero_slice(zbuf, [deg_sp], sid)
    plsc.subcore_barrier()

    def load(j):
        b = j % NB
        base = wid * EPT + j * W
        pltpu.async_copy(dst_hbm.at[pl.ds(base, W)], dst_v[b], sem_l[b])
        pltpu.async_copy(w_hbm.at[pl.ds(base, W)], w_v[b], sem_l[b])

    def wait_load(j):
        b = j % NB
        base = wid * EPT + j * W
        pltpu.make_async_copy(dst_hbm.at[pl.ds(base, W)], dst_v[b], sem_l[b]).wait()
        pltpu.make_async_copy(w_hbm.at[pl.ds(base, W)], w_v[b], sem_l[b]).wait()

    def scat(j):
        b = j % NB
        pltpu.async_copy(w_v[b], deg_sp.at[dst_v[b]], sem_s[b], add=True)

    def wait_scat(j):
        b = j % NB
        pltpu.make_async_copy(w_v[b], deg_sp.at[dst_v[b]], sem_s[b]).wait()

    load(0)
    load(1)
    for j in range(NWIN):
        wait_load(j)
        if j >= 2:
            wait_scat(j - 2)
        if j + 2 < NWIN:
            load(j + 2)
        scat(j)
    wait_scat(NWIN - 2)
    wait_scat(NWIN - 1)
    plsc.subcore_barrier()
    sl = pl.ds(sid * SLICE, SLICE)
    pltpu.sync_copy(deg_sp.at[sl], zbuf)
    pltpu.sync_copy(zbuf, out_hbm.at[pl.ds(cid * NPAD + sid * SLICE, SLICE)])


_deg_call = functools.partial(
    pl.kernel,
    out_type=jax.ShapeDtypeStruct((NC * NPAD,), jnp.float32),
    mesh=_sc_mesh(),
    scratch_types=[
        [pltpu.VMEM((W,), jnp.int32) for _ in range(NB)],
        [pltpu.VMEM((W,), jnp.float32) for _ in range(NB)],
        pltpu.VMEM((SLICE,), jnp.float32),
        pltpu.VMEM_SHARED((NPAD,), jnp.float32),
        [pltpu.SemaphoreType.DMA for _ in range(NB)],
        [pltpu.SemaphoreType.DMA for _ in range(NB)],
    ],
)(_deg_body)


def _k2_body(degp_ref, xt_ref, dinv_ref, yt_ref):
    deg = degp_ref[0:1, :] + degp_ref[1:2, :] + 1.0
    dinv = jnp.where(deg > 0.0, lax.rsqrt(deg), 0.0)
    dinv_ref[...] = dinv
    yt_ref[...] = xt_ref[...] * dinv


_k2_call = pl.pallas_call(
    _k2_body,
    out_shape=[
        jax.ShapeDtypeStruct((1, NPAD), jnp.float32),
        jax.ShapeDtypeStruct((4, NPAD), jnp.float32),
    ],
)


def _agg_body(src_hbm, dst_hbm, w_hbm, yt_hbm, out_hbm,
              src_v, dst_v, w_v, r, zbuf, y, a,
              sem_l, sem_g, sem_s):
    cid = lax.axis_index("c")
    sid = lax.axis_index("s")
    wid = cid * NS + sid
    sl = pl.ds(sid * SLICE, SLICE)
    for k in range(4):
        pltpu.sync_copy(yt_hbm.at[pl.ds(k * NPAD + sid * SLICE, SLICE)], zbuf)
        pltpu.sync_copy(zbuf, y[k].at[sl])
    _zero_slice(zbuf, a, sid)
    plsc.subcore_barrier()

    def load(j):
        b = j % NB
        base = wid * EPT + j * W
        pltpu.async_copy(src_hbm.at[pl.ds(base, W)], src_v[b], sem_l[b])
        pltpu.async_copy(dst_hbm.at[pl.ds(base, W)], dst_v[b], sem_l[b])
        pltpu.async_copy(w_hbm.at[pl.ds(base, W)], w_v[b], sem_l[b])

    def wait_load(j):
        b = j % NB
        base = wid * EPT + j * W
        pltpu.make_async_copy(src_hbm.at[pl.ds(base, W)], src_v[b], sem_l[b]).wait()
        pltpu.make_async_copy(dst_hbm.at[pl.ds(base, W)], dst_v[b], sem_l[b]).wait()
        pltpu.make_async_copy(w_hbm.at[pl.ds(base, W)], w_v[b], sem_l[b]).wait()

    def wait_scat(j):
        b = j % 2
        for k in range(4):
            pltpu.make_async_copy(r[b][k], a[k].at[dst_v[j % NB]], sem_s[b]).wait()

    load(0)
    load(1)
    for j in range(NWIN):
        bi = j % NB
        br = j % 2
        wait_load(j)
        if j >= 2:
            wait_scat(j - 2)
        if j + 2 < NWIN:
            load(j + 2)
        for k in range(4):
            pltpu.async_copy(y[k].at[src_v[bi]], r[br][k], sem_g[br])
        for k in range(4):
            pltpu.make_async_copy(y[k].at[src_v[bi]], r[br][k], sem_g[br]).wait()

        def mul(i, _c):
            off = pl.ds(i * 16, 16)
            wv = w_v[bi][off]
            for k in range(4):
                r[br][k][off] = r[br][k][off] * wv
            return _c

        lax.fori_loop(0, W // 16, mul, None)
        for k in range(4):
            pltpu.async_copy(r[br][k], a[k].at[dst_v[bi]], sem_s[br], add=True)
    wait_scat(NWIN - 2)
    wait_scat(NWIN - 1)
    plsc.subcore_barrier()
    for k in range(4):
        pltpu.sync_copy(a[k].at[sl], zbuf)
        pltpu.sync_copy(zbuf, out_hbm.at[pl.ds((cid * 4 + k) * NPAD + sid * SLICE, SLICE)])


_agg_call = functools.partial(
    pl.kernel,
    out_type=jax.ShapeDtypeStruct((NC * 4 * NPAD,), jnp.float32),
    mesh=_sc_mesh(),
    scratch_types=[
        [pltpu.VMEM((W,), jnp.int32) for _ in range(NB)],
        [pltpu.VMEM((W,), jnp.int32) for _ in range(NB)],
        [pltpu.VMEM((W,), jnp.float32) for _ in range(NB)],
        [[pltpu.VMEM((W,), jnp.float32) for _ in range(4)] for _ in range(2)],
        pltpu.VMEM((SLICE,), jnp.float32),
        [pltpu.VMEM_SHARED((NPAD,), jnp.float32) for _ in range(4)],
        [pltpu.VMEM_SHARED((NPAD,), jnp.float32) for _ in range(4)],
        [pltpu.SemaphoreType.DMA for _ in range(NB)],
        [pltpu.SemaphoreType.DMA for _ in range(2)],
        [pltpu.SemaphoreType.DMA for _ in range(2)],
    ],
)(_agg_body)


BN = 2048


def _k4_body(aggp_ref, dinv_ref, yt_ref, wc_ref, bc_ref, out_ref):
    pre = (aggp_ref[0] + aggp_ref[1] + yt_ref[...]) * dinv_ref[...]
    out_ref[...] = (
        lax.dot_general(pre, wc_ref[...], (((0,), (0,)), ((), ())),
                        preferred_element_type=jnp.float32)
        + bc_ref[...]
    )


_k4_call = pl.pallas_call(
    _k4_body,
    grid=(pl.cdiv(N, BN),),
    in_specs=[
        pl.BlockSpec((NC, 4, BN), lambda i: (0, 0, i)),
        pl.BlockSpec((1, BN), lambda i: (0, i)),
        pl.BlockSpec((4, BN), lambda i: (0, i)),
        pl.BlockSpec((4, 48), lambda i: (0, 0)),
        pl.BlockSpec((1, 48), lambda i: (0, 0)),
    ],
    out_specs=pl.BlockSpec((BN, 48), lambda i: (i, 0)),
    out_shape=jax.ShapeDtypeStruct((N, 48), jnp.float32),
)


def kernel(x, edge_index, edge_attr, W1, b1, W2, b2, W3, b3):
    src = edge_index[0]
    dst = edge_index[1]
    w = edge_attr[:, 0]
    xt = jnp.pad(x.T, ((0, 0), (0, NPAD - N)))
    wc = jnp.concatenate([W1, W2, W3], axis=1)
    bc = jnp.concatenate([b1, b2, b3])[None, :]

    degp = _deg_call(dst, w).reshape(NC, NPAD)
    dinv, yt = _k2_call(degp, xt)
    aggp = _agg_call(src, dst, w, yt.reshape(4 * NPAD)).reshape(NC, 4, NPAD)
    return _k4_call(aggp, dinv, yt, wc, bc)
